# Initial kernel scaffold; baseline (speedup 1.0000x reference)
import jax
import jax.numpy as jnp
from jax.experimental import pallas as pl


def kernel(x, special_tokens_indices, W):
    raise NotImplementedError("write your pallas kernel here")


# speedup vs baseline: 1.6217x; 1.6217x over previous
import functools

import jax
import jax.numpy as jnp
from jax.experimental import pallas as pl
from jax.experimental.pallas import tpu as pltpu

_NUM_SPECIAL = 16
_BLOCK_S = 512


def _body(idx_ref, w_ref, o_ref):
    bs = o_ref.shape[1]
    base = pl.program_id(1) * bs
    pos = jax.lax.broadcasted_iota(jnp.int32, (bs, 1), 0) + base
    mask = jnp.zeros((bs, 1), dtype=jnp.bool_)
    for j in range(_NUM_SPECIAL):
        mask = jnp.logical_or(mask, pos == idx_ref[j])
    rows = jnp.where(mask, w_ref[1], w_ref[0])
    o_ref[...] = rows[None]


def kernel(x, special_tokens_indices, W):
    B, S, H = x.shape
    idx = special_tokens_indices.astype(jnp.int32)
    grid = (B, S // _BLOCK_S)
    out = pl.pallas_call(
        _body,
        grid=grid,
        in_specs=[
            pl.BlockSpec(memory_space=pltpu.SMEM),
            pl.BlockSpec((2, H), lambda b, s: (0, 0)),
        ],
        out_specs=pl.BlockSpec((1, _BLOCK_S, H), lambda b, s: (b, s, 0)),
        out_shape=jax.ShapeDtypeStruct((B, S, H), jnp.float32),
        compiler_params=pltpu.CompilerParams(
            dimension_semantics=("parallel", "parallel"),
        ),
    )(idx, W)
    return out


# speedup vs baseline: 2.9276x; 1.8053x over previous
import functools

import jax
import jax.numpy as jnp
from jax.experimental import pallas as pl
from jax.experimental.pallas import tpu as pltpu

_NUM_SPECIAL = 16
_BLOCK_S = 512


def _body(idx_ref, w_ref, o_ref):
    nb = o_ref.shape[0]
    bs = o_ref.shape[1]
    base = pl.program_id(0) * bs
    pos = jax.lax.broadcasted_iota(jnp.int32, (bs, 1), 0) + base
    mask = jnp.zeros((bs, 1), dtype=jnp.bool_)
    for j in range(_NUM_SPECIAL):
        mask = jnp.logical_or(mask, pos == idx_ref[j])
    rows = jnp.where(mask, w_ref[1], w_ref[0])
    for b in range(nb):
        o_ref[b] = rows


def kernel(x, special_tokens_indices, W):
    B, S, H = x.shape
    idx = special_tokens_indices.astype(jnp.int32)
    grid = (S // _BLOCK_S,)
    out = pl.pallas_call(
        _body,
        grid=grid,
        in_specs=[
            pl.BlockSpec(memory_space=pltpu.SMEM),
            pl.BlockSpec((2, H), lambda s: (0, 0)),
        ],
        out_specs=pl.BlockSpec((B, _BLOCK_S, H), lambda s: (0, s, 0)),
        out_shape=jax.ShapeDtypeStruct((B, S, H), jnp.float32),
        compiler_params=pltpu.CompilerParams(
            dimension_semantics=("arbitrary",),
        ),
    )(idx, W)
    return out
